# baseline (speedup 1.0000x reference)
import jax
import jax.numpy as jnp
from jax.experimental import pallas as pl
from jax.experimental.pallas import tpu as pltpu

CHANNELS = 768
TOPK = 384
TEMP = 1.0
C_BLK = 128
N_CBLK = CHANNELS // C_BLK
NB = 16
XDIM = 56 * 56


def _mask_kernel(logits_ref, m_ref, meta_ref):
    lg = logits_ref[0, :]
    col = lg[None, :]
    row = lg[:, None]
    i_idx = jax.lax.broadcasted_iota(jnp.int32, (CHANNELS, CHANNELS), 0)
    j_idx = jax.lax.broadcasted_iota(jnp.int32, (CHANNELS, CHANNELS), 1)
    beats = (col > row) | ((col == row) & (j_idx < i_idx))
    rank = jnp.sum(beats.astype(jnp.int32), axis=1)
    hard = (rank < TOPK).astype(jnp.float32)
    soft = jax.nn.sigmoid(lg / TEMP)
    m = (hard - soft) + soft
    m_ref[0, :] = m

    act = (jnp.sum(hard.reshape(N_CBLK, C_BLK), axis=1) > 0).astype(jnp.int32)
    a_col = act[None, :]
    ci = jax.lax.broadcasted_iota(jnp.int32, (N_CBLK, N_CBLK), 0)
    cj = jax.lax.broadcasted_iota(jnp.int32, (N_CBLK, N_CBLK), 1)
    inc = jnp.sum(jnp.where(cj <= ci, a_col, 0), axis=1)
    num_active = jnp.sum(act)
    c_lin = inc[0] * 0 + jax.lax.broadcasted_iota(jnp.int32, (1, N_CBLK), 1)[0]
    pos = jnp.where(act == 1, inc - 1, num_active + c_lin - inc)
    perm = jnp.sum(jnp.where(pos[None, :] == ci, cj, 0), axis=1)
    last_active = jnp.sum(jnp.where(c_lin == num_active - 1, perm, 0))
    p_lin = c_lin
    zidx = jnp.where(p_lin < num_active, perm, last_active)

    lane = jax.lax.broadcasted_iota(jnp.int32, (1, 128), 1)
    c_sub = jax.lax.broadcasted_iota(jnp.int32, (N_CBLK, 128), 0)
    lane2 = jax.lax.broadcasted_iota(jnp.int32, (N_CBLK, 128), 1)
    meta = (jnp.sum(jnp.where(lane2 == c_sub, zidx[:, None], 0), axis=0)
            + jnp.sum(jnp.where(lane2 == c_sub + 8, perm[:, None], 0), axis=0))
    meta = meta + jnp.where(lane[0] == 7, num_active, 0)
    meta_ref[0, :] = meta


def _gate_kernel(meta_ref, z_ref, m_ref, out_ref):
    del meta_ref
    out_ref[0] = z_ref[0] * m_ref[0]


def kernel(z, logits):
    z3 = z.reshape(NB, CHANNELS, XDIM)
    m_out, meta = pl.pallas_call(
        _mask_kernel,
        out_shape=(
            jax.ShapeDtypeStruct((1, CHANNELS), jnp.float32),
            jax.ShapeDtypeStruct((1, 128), jnp.int32),
        ),
    )(logits.reshape(1, CHANNELS))
    m3 = m_out.reshape(N_CBLK, C_BLK, 1)

    def z_map(c, b, meta):
        return (jnp.where(c < meta[0, 7], b, NB - 1), meta[0, c], 0)

    def m_map(c, b, meta):
        return (meta[0, 8 + c], 0, 0)

    def out_map(c, b, meta):
        return (b, meta[0, 8 + c], 0)

    grid_spec = pltpu.PrefetchScalarGridSpec(
        num_scalar_prefetch=1,
        grid=(N_CBLK, NB),
        in_specs=[
            pl.BlockSpec((1, C_BLK, XDIM), z_map),
            pl.BlockSpec((1, C_BLK, 1), m_map),
        ],
        out_specs=pl.BlockSpec((1, C_BLK, XDIM), out_map),
    )
    out = pl.pallas_call(
        _gate_kernel,
        grid_spec=grid_spec,
        out_shape=jax.ShapeDtypeStruct((NB, CHANNELS, XDIM), jnp.float32),
    )(meta, z3, m3)
    return out.reshape(z.shape)
